# baseline (speedup 1.0000x reference)
import functools

import jax
import jax.numpy as jnp
from jax import lax
from jax.experimental import pallas as pl
from jax.experimental.pallas import tpu as pltpu
from jax.experimental.pallas import tpu_sc as plsc

_N = 10000
_E = 320000
_D = 128
_H = _D // 2
_C = 40

_NC = 2
_NS = 16
_NW = _NC * _NS
_K = 80
_G = 5
_RPS = 624
_RTL = _N - _NS * _RPS

_BN = 1000


def _zero_rows(sid, zrows, acc):
    pltpu.sync_copy(zrows.at[pl.ds(sid * _RPS, _RPS)],
                    acc.at[pl.ds(sid * _RPS, _RPS)])

    @pl.when(sid == 0)
    def _():
        pltpu.sync_copy(zrows.at[pl.ds(_NS * _RPS, _RTL)],
                        acc.at[pl.ds(_NS * _RPS, _RTL)])


def _copy_out_rows(sid, acc, out):
    pltpu.sync_copy(acc.at[pl.ds(sid * _RPS, _RPS)],
                    out.at[pl.ds(sid * _RPS, _RPS)])

    @pl.when(sid == 0)
    def _():
        pltpu.sync_copy(acc.at[pl.ds(_NS * _RPS, _RTL)],
                        out.at[pl.ds(_NS * _RPS, _RTL)])


def _make_sc_pass(d, with_cnt):
    eps = _E // _NS if with_cnt else _E // _NW
    nhalf = 2 if with_cnt else 1
    seps = eps // nhalf
    nch = seps // _K
    ngr = nch // _G
    assert ngr * _G == nch and ngr % 2 == 1

    mesh = plsc.VectorSubcoreMesh(core_axis_name="c", subcore_axis_name="s",
                                  num_cores=_NC, num_subcores=_NS)
    out_type = [jax.ShapeDtypeStruct((_NC, _N, d), jnp.float32)]
    if with_cnt:
        out_type.append(jax.ShapeDtypeStruct((_NC, _N), jnp.float32))
    scratch = [
        pltpu.VMEM((seps,), jnp.int32),
        pltpu.VMEM((seps,), jnp.int32),
        pltpu.VMEM((_K,), jnp.float32),
        pltpu.VMEM_SHARED((_N, d), jnp.float32),
        pltpu.VMEM_SHARED((_N,), jnp.float32),
    ]
    if not with_cnt:
        scratch.append(pltpu.VMEM_SHARED((_N, d), jnp.float32))
    for _ in range(2 * _G):
        scratch += [pltpu.VMEM((_K,), jnp.int32),
                    pltpu.VMEM((_K,), jnp.int32),
                    pltpu.VMEM((_K, d), jnp.float32)]
    scratch += [pltpu.SemaphoreType.DMA] * 4

    def body(*refs):
        if with_cnt:
            (table, eidx, zrows, zn, outs, outc) = refs[:6]
            rest = refs[6:]
        else:
            (table, eidx, zrows, outs) = refs[:4]
            rest = refs[4:]
        srcv, dstv, ones, acc, accc = rest[:5]
        if with_cnt:
            gtab = table
            rest = rest[5:]
        else:
            gtab = rest[5]
            rest = rest[6:]
        banks = [tuple(rest[3 * i: 3 + 3 * i]) for i in range(2 * _G)]
        pool_a, pool_b = banks[:_G], banks[_G:]
        gsa, gsb, ssa, ssb = rest[6 * _G: 4 + 6 * _G]

        cid = lax.axis_index("c")
        sid = lax.axis_index("s")
        if with_cnt:
            base = sid * eps
        else:
            base = (sid * _NC + cid) * eps

        _zero_rows(sid, zrows, acc)
        if with_cnt:
            @pl.when(sid == 1)
            def _():
                pltpu.sync_copy(zn, accc)
            for i in range(_K // 16):
                ones[pl.ds(i * 16, 16)] = jnp.ones((16,), jnp.float32)
        else:
            pltpu.sync_copy(table.at[pl.ds(sid * _RPS, _RPS)],
                            gtab.at[pl.ds(sid * _RPS, _RPS)])

            @pl.when(sid == 1)
            def _():
                pltpu.sync_copy(table.at[pl.ds(_NS * _RPS, _RTL)],
                                gtab.at[pl.ds(_NS * _RPS, _RTL)])

        plsc.subcore_barrier()

        def fire_gathers(g, pool, gsem):
            for b in range(_G):
                ibuf, dbuf, rbuf = pool[b]
                off = (g * _G + b) * _K
                for i in range(_K // 16):
                    s = srcv[pl.ds(off + i * 16, 16)]
                    if with_cnt:
                        s = s * 2 + cid
                    ibuf[pl.ds(i * 16, 16)] = s
                    dbuf[pl.ds(i * 16, 16)] = dstv[pl.ds(off + i * 16, 16)]
                pltpu.async_copy(gtab.at[ibuf], rbuf, gsem)

        def drain_gathers(pool, gsem):
            for b in range(_G):
                ibuf, _, rbuf = pool[b]
                pltpu.make_async_copy(gtab.at[ibuf], rbuf, gsem).wait()

        def fire_scatters(pool, ssem, cpred):
            for b in range(_G):
                _, dbuf, rbuf = pool[b]
                pltpu.async_copy(rbuf, acc.at[dbuf], ssem, add=True)
                if with_cnt:
                    @pl.when(cpred)
                    def _():
                        pltpu.async_copy(ones, accc.at[dbuf], ssem, add=True)

        def drain_scatters(pool, ssem, cpred):
            for b in range(_G):
                _, dbuf, rbuf = pool[b]
                pltpu.make_async_copy(rbuf, acc.at[dbuf], ssem).wait()
                if with_cnt:
                    @pl.when(cpred)
                    def _():
                        pltpu.make_async_copy(ones, accc.at[dbuf], ssem).wait()

        for h in range(nhalf):
            cpred = (cid == h)

            def it(v, carry, cpred=cpred):
                drain_gathers(pool_a, gsa)

                @pl.when(v > 0)
                def _():
                    drain_scatters(pool_b, ssb, cpred)

                fire_gathers(2 * v + 1, pool_b, gsb)
                fire_scatters(pool_a, ssa, cpred)
                drain_gathers(pool_b, gsb)
                drain_scatters(pool_a, ssa, cpred)

                @pl.when(2 * v + 2 < ngr)
                def _():
                    fire_gathers(2 * v + 2, pool_a, gsa)

                fire_scatters(pool_b, ssb, cpred)
                return carry

            pltpu.sync_copy(eidx.at[0, pl.ds(base + h * seps, seps)], srcv)
            pltpu.sync_copy(eidx.at[1, pl.ds(base + h * seps, seps)], dstv)
            fire_gathers(0, pool_a, gsa)
            lax.fori_loop(0, ngr // 2, it, 0)
            drain_gathers(pool_a, gsa)
            drain_scatters(pool_b, ssb, cpred)
            fire_scatters(pool_a, ssa, cpred)
            drain_scatters(pool_a, ssa, cpred)
        plsc.subcore_barrier()

        _copy_out_rows(sid, acc, outs.at[cid])
        if with_cnt:
            @pl.when(sid == 1)
            def _():
                pltpu.sync_copy(accc, outc.at[cid])

    return pl.kernel(body, out_type=out_type, mesh=mesh, scratch_types=scratch,
                     compiler_params=pltpu.CompilerParams(use_tc_tiling_on_sc=False))


_sc_pass_cached = functools.lru_cache(maxsize=None)(_make_sc_pass)


def _tc_proj(x, wl, wr, b):
    def body(x_ref, wl_ref, wr_ref, b_ref, p_ref, r_ref):
        xb = x_ref[...]
        p_ref[...] = jnp.dot(xb, wl_ref[...], preferred_element_type=jnp.float32)
        r_ref[...] = (jnp.dot(xb, wr_ref[...], preferred_element_type=jnp.float32)
                      + b_ref[...])
    return pl.pallas_call(
        body,
        grid=(_N // _BN,),
        in_specs=[pl.BlockSpec((_BN, _D), lambda i: (i, 0)),
                  pl.BlockSpec((_D, _D), lambda i: (0, 0)),
                  pl.BlockSpec((_D, _D), lambda i: (0, 0)),
                  pl.BlockSpec((1, _D), lambda i: (0, 0))],
        out_specs=[pl.BlockSpec((_BN, _D), lambda i: (i, 0)),
                   pl.BlockSpec((_BN, _D), lambda i: (i, 0))],
        out_shape=[jax.ShapeDtypeStruct((_N, _D), jnp.float32)] * 2,
    )(x, wl, wr, b.reshape(1, _D))


def _tc_mid(s0, cnt, r0, wl1, wr1, bl1):
    def body(sa_ref, sb_ref, ca_ref, cb_ref, r0_ref, wl_ref, wr_ref, b_ref,
             q_ref, r_ref):
        inv = 1.0 / jnp.maximum(ca_ref[0] + cb_ref[0], 1.0)
        r0b = r0_ref[...]
        ha = jnp.maximum(sa_ref[0] * inv + r0b[:, :_H], 0.0)
        hb = jnp.maximum(sb_ref[0] * inv + r0b[:, _H:], 0.0)
        wl = wl_ref[...]
        wr = wr_ref[...]
        q_ref[...] = (
            jnp.dot(ha, wl[:_H], preferred_element_type=jnp.float32)
            + jnp.dot(hb, wl[_H:], preferred_element_type=jnp.float32))
        r_ref[...] = (
            jnp.dot(ha, wr[:_H], preferred_element_type=jnp.float32)
            + jnp.dot(hb, wr[_H:], preferred_element_type=jnp.float32)
            + b_ref[...])
    return pl.pallas_call(
        body,
        grid=(_N // _BN,),
        in_specs=[pl.BlockSpec((1, _BN, _H), lambda i: (0, i, 0)),
                  pl.BlockSpec((1, _BN, _H), lambda i: (1, i, 0)),
                  pl.BlockSpec((1, _BN, 1), lambda i: (0, i, 0)),
                  pl.BlockSpec((1, _BN, 1), lambda i: (1, i, 0)),
                  pl.BlockSpec((_BN, _D), lambda i: (i, 0)),
                  pl.BlockSpec((_D, _C), lambda i: (0, 0)),
                  pl.BlockSpec((_D, _C), lambda i: (0, 0)),
                  pl.BlockSpec((1, _C), lambda i: (0, 0))],
        out_specs=[pl.BlockSpec((_BN, _C), lambda i: (i, 0)),
                   pl.BlockSpec((_BN, _C), lambda i: (i, 0))],
        out_shape=[jax.ShapeDtypeStruct((_N, _C), jnp.float32)] * 2,
    )(s0, s0, cnt, cnt, r0, wl1, wr1, bl1.reshape(1, _C))


def _tc_final(s1, cnt, r1):
    def body(s_ref, ca_ref, cb_ref, r_ref, o_ref, g_ref):
        s = s_ref[0] + s_ref[1]
        o = s / jnp.maximum(ca_ref[0] + cb_ref[0], 1.0) + r_ref[...]
        o_ref[...] = o

        @pl.when(pl.program_id(0) == 0)
        def _():
            g_ref[...] = jnp.zeros_like(g_ref)
        g_ref[...] += jnp.sum(o, axis=0, keepdims=True) * (1.0 / _N)
    return pl.pallas_call(
        body,
        grid=(_N // _BN,),
        in_specs=[pl.BlockSpec((_NC, _BN, _C), lambda i: (0, i, 0)),
                  pl.BlockSpec((1, _BN, 1), lambda i: (0, i, 0)),
                  pl.BlockSpec((1, _BN, 1), lambda i: (1, i, 0)),
                  pl.BlockSpec((_BN, _C), lambda i: (i, 0))],
        out_specs=[pl.BlockSpec((_BN, _C), lambda i: (i, 0)),
                   pl.BlockSpec((1, _C), lambda i: (0, 0))],
        out_shape=[jax.ShapeDtypeStruct((_N, _C), jnp.float32),
                   jax.ShapeDtypeStruct((1, _C), jnp.float32)],
    )(s1, cnt, cnt, r1)


def kernel(x, edge_index, Wl0, bl0, Wr0, Wl1, bl1, Wr1):
    z0 = jnp.zeros((_N, _H), jnp.float32)
    z1 = jnp.zeros((_N, _C), jnp.float32)
    zn = jnp.zeros((_N,), jnp.float32)

    p0, r0 = _tc_proj(x, Wl0, Wr0, bl0)
    p0f = p0.reshape(_NC * _N, _H)
    s0, cnt = _sc_pass_cached(_H, True)(p0f, edge_index, z0, zn)
    cnt = cnt.reshape(_NC, _N, 1)
    q1, r1 = _tc_mid(s0, cnt, r0, Wl1, Wr1, bl1)
    s1 = _sc_pass_cached(_C, False)(q1, edge_index, z1)
    if isinstance(s1, (list, tuple)):
        (s1,) = s1
    out, g = _tc_final(s1, cnt, r1)
    return (out, g)


# speedup vs baseline: 1.0035x; 1.0035x over previous
import functools

import jax
import jax.numpy as jnp
from jax import lax
from jax.experimental import pallas as pl
from jax.experimental.pallas import tpu as pltpu
from jax.experimental.pallas import tpu_sc as plsc

_N = 10000
_E = 320000
_D = 128
_H = _D // 2
_C = 40

_NC = 2
_NS = 16
_NW = _NC * _NS
_K = 80
_G = 5
_RPS = 624
_RTL = _N - _NS * _RPS

_BN = 1000
_SP_TABLE = False


def _zero_rows(sid, zrows, acc):
    pltpu.sync_copy(zrows.at[pl.ds(sid * _RPS, _RPS)],
                    acc.at[pl.ds(sid * _RPS, _RPS)])

    @pl.when(sid == 0)
    def _():
        pltpu.sync_copy(zrows.at[pl.ds(_NS * _RPS, _RTL)],
                        acc.at[pl.ds(_NS * _RPS, _RTL)])


def _copy_out_rows(sid, acc, out):
    pltpu.sync_copy(acc.at[pl.ds(sid * _RPS, _RPS)],
                    out.at[pl.ds(sid * _RPS, _RPS)])

    @pl.when(sid == 0)
    def _():
        pltpu.sync_copy(acc.at[pl.ds(_NS * _RPS, _RTL)],
                        out.at[pl.ds(_NS * _RPS, _RTL)])


def _make_sc_pass(d, with_cnt):
    eps = _E // _NS if with_cnt else _E // _NW
    nhalf = 2 if with_cnt else 1
    seps = eps // nhalf
    nch = seps // _K
    ngr = nch // _G
    assert ngr * _G == nch and ngr % 2 == 1

    mesh = plsc.VectorSubcoreMesh(core_axis_name="c", subcore_axis_name="s",
                                  num_cores=_NC, num_subcores=_NS)
    out_type = [jax.ShapeDtypeStruct((_NC, _N, d), jnp.float32)]
    if with_cnt:
        out_type.append(jax.ShapeDtypeStruct((_NC, _N), jnp.float32))
    scratch = [
        pltpu.VMEM((seps,), jnp.int32),
        pltpu.VMEM((seps,), jnp.int32),
        pltpu.VMEM((_K,), jnp.float32),
        pltpu.VMEM_SHARED((_N, d), jnp.float32),
        pltpu.VMEM_SHARED((_N,), jnp.float32),
    ]
    if not with_cnt:
        scratch.append(pltpu.VMEM_SHARED((_N, d), jnp.float32))
    for _ in range(2 * _G):
        scratch += [pltpu.VMEM((_K,), jnp.int32),
                    pltpu.VMEM((_K,), jnp.int32),
                    pltpu.VMEM((_K, d), jnp.float32)]
    scratch += [pltpu.SemaphoreType.DMA] * 4

    def body(*refs):
        if with_cnt:
            (table, eidx, zrows, zn, outs, outc) = refs[:6]
            rest = refs[6:]
        else:
            (table, eidx, zrows, outs) = refs[:4]
            rest = refs[4:]
        srcv, dstv, ones, acc, accc = rest[:5]
        if with_cnt:
            gtab = table
            rest = rest[5:]
        else:
            sptab = rest[5]
            gtab = sptab if _SP_TABLE else table
            rest = rest[6:]
        banks = [tuple(rest[3 * i: 3 + 3 * i]) for i in range(2 * _G)]
        pool_a, pool_b = banks[:_G], banks[_G:]
        gsa, gsb, ssa, ssb = rest[6 * _G: 4 + 6 * _G]

        cid = lax.axis_index("c")
        sid = lax.axis_index("s")
        if with_cnt:
            base = sid * eps
        else:
            base = (sid * _NC + cid) * eps

        _zero_rows(sid, zrows, acc)
        if with_cnt:
            @pl.when(sid == 1)
            def _():
                pltpu.sync_copy(zn, accc)
            for i in range(_K // 16):
                ones[pl.ds(i * 16, 16)] = jnp.ones((16,), jnp.float32)
        elif _SP_TABLE:
            pltpu.sync_copy(table.at[pl.ds(sid * _RPS, _RPS)],
                            gtab.at[pl.ds(sid * _RPS, _RPS)])

            @pl.when(sid == 1)
            def _():
                pltpu.sync_copy(table.at[pl.ds(_NS * _RPS, _RTL)],
                                gtab.at[pl.ds(_NS * _RPS, _RTL)])

        plsc.subcore_barrier()

        def fire_gathers(g, pool, gsem):
            for b in range(_G):
                ibuf, dbuf, rbuf = pool[b]
                off = (g * _G + b) * _K
                for i in range(_K // 16):
                    s = srcv[pl.ds(off + i * 16, 16)]
                    if with_cnt:
                        s = s * 2 + cid
                    ibuf[pl.ds(i * 16, 16)] = s
                    dbuf[pl.ds(i * 16, 16)] = dstv[pl.ds(off + i * 16, 16)]
                pltpu.async_copy(gtab.at[ibuf], rbuf, gsem)

        def drain_gathers(pool, gsem):
            for b in range(_G):
                ibuf, _, rbuf = pool[b]
                pltpu.make_async_copy(gtab.at[ibuf], rbuf, gsem).wait()

        def fire_scatters(pool, ssem, cpred):
            for b in range(_G):
                _, dbuf, rbuf = pool[b]
                pltpu.async_copy(rbuf, acc.at[dbuf], ssem, add=True)
                if with_cnt:
                    @pl.when(cpred)
                    def _():
                        pltpu.async_copy(ones, accc.at[dbuf], ssem, add=True)

        def drain_scatters(pool, ssem, cpred):
            for b in range(_G):
                _, dbuf, rbuf = pool[b]
                pltpu.make_async_copy(rbuf, acc.at[dbuf], ssem).wait()
                if with_cnt:
                    @pl.when(cpred)
                    def _():
                        pltpu.make_async_copy(ones, accc.at[dbuf], ssem).wait()

        for h in range(nhalf):
            cpred = (cid == h)

            def it(v, carry, cpred=cpred):
                drain_gathers(pool_a, gsa)

                @pl.when(v > 0)
                def _():
                    drain_scatters(pool_b, ssb, cpred)

                fire_gathers(2 * v + 1, pool_b, gsb)
                fire_scatters(pool_a, ssa, cpred)
                drain_gathers(pool_b, gsb)
                drain_scatters(pool_a, ssa, cpred)

                @pl.when(2 * v + 2 < ngr)
                def _():
                    fire_gathers(2 * v + 2, pool_a, gsa)

                fire_scatters(pool_b, ssb, cpred)
                return carry

            pltpu.sync_copy(eidx.at[0, pl.ds(base + h * seps, seps)], srcv)
            pltpu.sync_copy(eidx.at[1, pl.ds(base + h * seps, seps)], dstv)
            fire_gathers(0, pool_a, gsa)
            lax.fori_loop(0, ngr // 2, it, 0)
            drain_gathers(pool_a, gsa)
            drain_scatters(pool_b, ssb, cpred)
            fire_scatters(pool_a, ssa, cpred)
            drain_scatters(pool_a, ssa, cpred)
        plsc.subcore_barrier()

        _copy_out_rows(sid, acc, outs.at[cid])
        if with_cnt:
            @pl.when(sid == 1)
            def _():
                pltpu.sync_copy(accc, outc.at[cid])

    return pl.kernel(body, out_type=out_type, mesh=mesh, scratch_types=scratch,
                     compiler_params=pltpu.CompilerParams(use_tc_tiling_on_sc=False))


_sc_pass_cached = functools.lru_cache(maxsize=None)(_make_sc_pass)


def _tc_proj(x, wl, wr, b):
    def body(x_ref, wl_ref, wr_ref, b_ref, p_ref, r_ref):
        xb = x_ref[...]
        p_ref[...] = jnp.dot(xb, wl_ref[...], preferred_element_type=jnp.float32)
        r_ref[...] = (jnp.dot(xb, wr_ref[...], preferred_element_type=jnp.float32)
                      + b_ref[...])
    return pl.pallas_call(
        body,
        grid=(_N // _BN,),
        in_specs=[pl.BlockSpec((_BN, _D), lambda i: (i, 0)),
                  pl.BlockSpec((_D, _D), lambda i: (0, 0)),
                  pl.BlockSpec((_D, _D), lambda i: (0, 0)),
                  pl.BlockSpec((1, _D), lambda i: (0, 0))],
        out_specs=[pl.BlockSpec((_BN, _D), lambda i: (i, 0)),
                   pl.BlockSpec((_BN, _D), lambda i: (i, 0))],
        out_shape=[jax.ShapeDtypeStruct((_N, _D), jnp.float32)] * 2,
    )(x, wl, wr, b.reshape(1, _D))


def _tc_mid(s0, cnt, r0, wl1, wr1, bl1):
    def body(sa_ref, sb_ref, ca_ref, cb_ref, r0_ref, wl_ref, wr_ref, b_ref,
             q_ref, r_ref):
        inv = 1.0 / jnp.maximum(ca_ref[0] + cb_ref[0], 1.0)
        r0b = r0_ref[...]
        ha = jnp.maximum(sa_ref[0] * inv + r0b[:, :_H], 0.0)
        hb = jnp.maximum(sb_ref[0] * inv + r0b[:, _H:], 0.0)
        wl = wl_ref[...]
        wr = wr_ref[...]
        q_ref[...] = (
            jnp.dot(ha, wl[:_H], preferred_element_type=jnp.float32)
            + jnp.dot(hb, wl[_H:], preferred_element_type=jnp.float32))
        r_ref[...] = (
            jnp.dot(ha, wr[:_H], preferred_element_type=jnp.float32)
            + jnp.dot(hb, wr[_H:], preferred_element_type=jnp.float32)
            + b_ref[...])
    return pl.pallas_call(
        body,
        grid=(_N // _BN,),
        in_specs=[pl.BlockSpec((1, _BN, _H), lambda i: (0, i, 0)),
                  pl.BlockSpec((1, _BN, _H), lambda i: (1, i, 0)),
                  pl.BlockSpec((1, _BN, 1), lambda i: (0, i, 0)),
                  pl.BlockSpec((1, _BN, 1), lambda i: (1, i, 0)),
                  pl.BlockSpec((_BN, _D), lambda i: (i, 0)),
                  pl.BlockSpec((_D, _C), lambda i: (0, 0)),
                  pl.BlockSpec((_D, _C), lambda i: (0, 0)),
                  pl.BlockSpec((1, _C), lambda i: (0, 0))],
        out_specs=[pl.BlockSpec((_BN, _C), lambda i: (i, 0)),
                   pl.BlockSpec((_BN, _C), lambda i: (i, 0))],
        out_shape=[jax.ShapeDtypeStruct((_N, _C), jnp.float32)] * 2,
    )(s0, s0, cnt, cnt, r0, wl1, wr1, bl1.reshape(1, _C))


def _tc_final(s1, cnt, r1):
    def body(s_ref, ca_ref, cb_ref, r_ref, o_ref, g_ref):
        s = s_ref[0] + s_ref[1]
        o = s / jnp.maximum(ca_ref[0] + cb_ref[0], 1.0) + r_ref[...]
        o_ref[...] = o

        @pl.when(pl.program_id(0) == 0)
        def _():
            g_ref[...] = jnp.zeros_like(g_ref)
        g_ref[...] += jnp.sum(o, axis=0, keepdims=True) * (1.0 / _N)
    return pl.pallas_call(
        body,
        grid=(_N // _BN,),
        in_specs=[pl.BlockSpec((_NC, _BN, _C), lambda i: (0, i, 0)),
                  pl.BlockSpec((1, _BN, 1), lambda i: (0, i, 0)),
                  pl.BlockSpec((1, _BN, 1), lambda i: (1, i, 0)),
                  pl.BlockSpec((_BN, _C), lambda i: (i, 0))],
        out_specs=[pl.BlockSpec((_BN, _C), lambda i: (i, 0)),
                   pl.BlockSpec((1, _C), lambda i: (0, 0))],
        out_shape=[jax.ShapeDtypeStruct((_N, _C), jnp.float32),
                   jax.ShapeDtypeStruct((1, _C), jnp.float32)],
    )(s1, cnt, cnt, r1)


def kernel(x, edge_index, Wl0, bl0, Wr0, Wl1, bl1, Wr1):
    z0 = jnp.zeros((_N, _H), jnp.float32)
    z1 = jnp.zeros((_N, _C), jnp.float32)
    zn = jnp.zeros((_N,), jnp.float32)

    p0, r0 = _tc_proj(x, Wl0, Wr0, bl0)
    p0f = p0.reshape(_NC * _N, _H)
    s0, cnt = _sc_pass_cached(_H, True)(p0f, edge_index, z0, zn)
    cnt = cnt.reshape(_NC, _N, 1)
    q1, r1 = _tc_mid(s0, cnt, r0, Wl1, Wr1, bl1)
    s1 = _sc_pass_cached(_C, False)(q1, edge_index, z1)
    if isinstance(s1, (list, tuple)):
        (s1,) = s1
    out, g = _tc_final(s1, cnt, r1)
    return (out, g)


# speedup vs baseline: 1.0405x; 1.0368x over previous
import functools

import jax
import jax.numpy as jnp
from jax import lax
from jax.experimental import pallas as pl
from jax.experimental.pallas import tpu as pltpu
from jax.experimental.pallas import tpu_sc as plsc

_N = 10000
_E = 320000
_D = 128
_H = _D // 2
_C = 40

_NC = 2
_NS = 16
_NW = _NC * _NS
_K = 80
_G = 5
_RPS = 624
_RTL = _N - _NS * _RPS

_BN = 1000


def _zero_rows(sid, zrows, acc):
    pltpu.sync_copy(zrows.at[pl.ds(sid * _RPS, _RPS)],
                    acc.at[pl.ds(sid * _RPS, _RPS)])

    @pl.when(sid == 0)
    def _():
        pltpu.sync_copy(zrows.at[pl.ds(_NS * _RPS, _RTL)],
                        acc.at[pl.ds(_NS * _RPS, _RTL)])


def _copy_out_rows(sid, acc, out):
    pltpu.sync_copy(acc.at[pl.ds(sid * _RPS, _RPS)],
                    out.at[pl.ds(sid * _RPS, _RPS)])

    @pl.when(sid == 0)
    def _():
        pltpu.sync_copy(acc.at[pl.ds(_NS * _RPS, _RTL)],
                        out.at[pl.ds(_NS * _RPS, _RTL)])


def _make_sc_pass(d, with_cnt):
    eps = _E // _NS if with_cnt else _E // _NW
    nhalf = 2 if with_cnt else 1
    seps = eps // nhalf
    nch = seps // _K
    ngr = nch // _G
    assert ngr * _G == nch and ngr % 2 == 1

    mesh = plsc.VectorSubcoreMesh(core_axis_name="c", subcore_axis_name="s",
                                  num_cores=_NC, num_subcores=_NS)
    out_type = [jax.ShapeDtypeStruct((_NC, _N, d), jnp.float32)]
    if with_cnt:
        out_type.append(jax.ShapeDtypeStruct((_N,), jnp.float32))
    scratch = [
        pltpu.VMEM((seps,), jnp.int32),
        pltpu.VMEM((seps,), jnp.int32),
        pltpu.VMEM((_K,), jnp.float32),
        pltpu.VMEM_SHARED((_N, d), jnp.float32),
        pltpu.VMEM_SHARED((_N,), jnp.float32),
    ]
    for _ in range(2 * _G):
        scratch += [pltpu.VMEM((_K,), jnp.int32),
                    pltpu.VMEM((_K,), jnp.int32),
                    pltpu.VMEM((_K, d), jnp.float32)]
    scratch += [pltpu.SemaphoreType.DMA] * 4

    def body(*refs):
        if with_cnt:
            (table, eidx, zrows, zn, outs, outc) = refs[:6]
            rest = refs[6:]
        else:
            (table, eidx, zrows, outs) = refs[:4]
            rest = refs[4:]
        srcv, dstv, ones, acc, accc = rest[:5]
        banks = [tuple(rest[5 + 3 * i: 8 + 3 * i]) for i in range(2 * _G)]
        pool_a, pool_b = banks[:_G], banks[_G:]
        gsa, gsb, ssa, ssb = rest[5 + 6 * _G: 9 + 6 * _G]

        cid = lax.axis_index("c")
        sid = lax.axis_index("s")
        if with_cnt:
            base = sid * eps
        else:
            base = (sid * _NC + cid) * eps

        _zero_rows(sid, zrows, acc)
        if with_cnt:
            @pl.when((cid == 0) & (sid == 1))
            def _():
                pltpu.sync_copy(zn, accc)
            for i in range(_K // 16):
                ones[pl.ds(i * 16, 16)] = jnp.ones((16,), jnp.float32)

        plsc.subcore_barrier()

        def fire_gathers(g, pool, gsem):
            for b in range(_G):
                ibuf, dbuf, rbuf = pool[b]
                off = (g * _G + b) * _K
                for i in range(_K // 16):
                    s = srcv[pl.ds(off + i * 16, 16)]
                    if with_cnt:
                        s = s * 2 + cid
                    ibuf[pl.ds(i * 16, 16)] = s
                    dbuf[pl.ds(i * 16, 16)] = dstv[pl.ds(off + i * 16, 16)]
                pltpu.async_copy(table.at[ibuf], rbuf, gsem)

        def drain_gathers(pool, gsem):
            for b in range(_G):
                ibuf, _, rbuf = pool[b]
                pltpu.make_async_copy(table.at[ibuf], rbuf, gsem).wait()

        def fire_scatters(pool, ssem):
            for b in range(_G):
                _, dbuf, rbuf = pool[b]
                pltpu.async_copy(rbuf, acc.at[dbuf], ssem, add=True)
                if with_cnt:
                    @pl.when(cid == 0)
                    def _():
                        pltpu.async_copy(ones, accc.at[dbuf], ssem, add=True)

        def drain_scatters(pool, ssem):
            for b in range(_G):
                _, dbuf, rbuf = pool[b]
                pltpu.make_async_copy(rbuf, acc.at[dbuf], ssem).wait()
                if with_cnt:
                    @pl.when(cid == 0)
                    def _():
                        pltpu.make_async_copy(ones, accc.at[dbuf], ssem).wait()

        def it(v, carry):
            drain_gathers(pool_a, gsa)

            @pl.when(v > 0)
            def _():
                drain_scatters(pool_b, ssb)

            fire_gathers(2 * v + 1, pool_b, gsb)
            fire_scatters(pool_a, ssa)
            drain_gathers(pool_b, gsb)
            drain_scatters(pool_a, ssa)

            @pl.when(2 * v + 2 < ngr)
            def _():
                fire_gathers(2 * v + 2, pool_a, gsa)

            fire_scatters(pool_b, ssb)
            return carry

        for h in range(nhalf):
            pltpu.sync_copy(eidx.at[0, pl.ds(base + h * seps, seps)], srcv)
            pltpu.sync_copy(eidx.at[1, pl.ds(base + h * seps, seps)], dstv)
            fire_gathers(0, pool_a, gsa)
            lax.fori_loop(0, ngr // 2, it, 0)
            drain_gathers(pool_a, gsa)
            drain_scatters(pool_b, ssb)
            fire_scatters(pool_a, ssa)
            drain_scatters(pool_a, ssa)
        plsc.subcore_barrier()

        _copy_out_rows(sid, acc, outs.at[cid])
        if with_cnt:
            @pl.when((cid == 0) & (sid == 1))
            def _():
                pltpu.sync_copy(accc, outc)

    return pl.kernel(body, out_type=out_type, mesh=mesh, scratch_types=scratch,
                     compiler_params=pltpu.CompilerParams(use_tc_tiling_on_sc=False))


_sc_pass_cached = functools.lru_cache(maxsize=None)(_make_sc_pass)


def _tc_proj(x, wl):
    def body(x_ref, wl_ref, p_ref):
        p_ref[...] = jnp.dot(x_ref[...], wl_ref[...],
                             preferred_element_type=jnp.float32)
    return pl.pallas_call(
        body,
        grid=(_N // _BN,),
        in_specs=[pl.BlockSpec((_BN, _D), lambda i: (i, 0)),
                  pl.BlockSpec((_D, _D), lambda i: (0, 0))],
        out_specs=pl.BlockSpec((_BN, _D), lambda i: (i, 0)),
        out_shape=jax.ShapeDtypeStruct((_N, _D), jnp.float32),
    )(x, wl)


def _tc_mid(s0, cnt, x, wr0, bl0, wl1, wr1, bl1):
    def body(sa_ref, sb_ref, c_ref, x_ref, wr0_ref, b0_ref,
             wl_ref, wr_ref, b_ref, q_ref, r_ref):
        inv = 1.0 / jnp.maximum(c_ref[...], 1.0)
        xb = x_ref[...]
        wr0 = wr0_ref[...]
        b0 = b0_ref[...]
        ra = (jnp.dot(xb, wr0[:, :_H], preferred_element_type=jnp.float32)
              + b0[:, :_H])
        rb = (jnp.dot(xb, wr0[:, _H:], preferred_element_type=jnp.float32)
              + b0[:, _H:])
        ha = jnp.maximum(sa_ref[0] * inv + ra, 0.0)
        hb = jnp.maximum(sb_ref[0] * inv + rb, 0.0)
        wl = wl_ref[...]
        wr = wr_ref[...]
        q_ref[...] = (
            jnp.dot(ha, wl[:_H], preferred_element_type=jnp.float32)
            + jnp.dot(hb, wl[_H:], preferred_element_type=jnp.float32))
        r_ref[...] = (
            jnp.dot(ha, wr[:_H], preferred_element_type=jnp.float32)
            + jnp.dot(hb, wr[_H:], preferred_element_type=jnp.float32)
            + b_ref[...])
    return pl.pallas_call(
        body,
        grid=(_N // _BN,),
        in_specs=[pl.BlockSpec((1, _BN, _H), lambda i: (0, i, 0)),
                  pl.BlockSpec((1, _BN, _H), lambda i: (1, i, 0)),
                  pl.BlockSpec((_BN, 1), lambda i: (i, 0)),
                  pl.BlockSpec((_BN, _D), lambda i: (i, 0)),
                  pl.BlockSpec((_D, _D), lambda i: (0, 0)),
                  pl.BlockSpec((1, _D), lambda i: (0, 0)),
                  pl.BlockSpec((_D, _C), lambda i: (0, 0)),
                  pl.BlockSpec((_D, _C), lambda i: (0, 0)),
                  pl.BlockSpec((1, _C), lambda i: (0, 0))],
        out_specs=[pl.BlockSpec((_BN, _C), lambda i: (i, 0)),
                   pl.BlockSpec((_BN, _C), lambda i: (i, 0))],
        out_shape=[jax.ShapeDtypeStruct((_N, _C), jnp.float32)] * 2,
    )(s0, s0, cnt, x, wr0, bl0.reshape(1, _D), wl1, wr1, bl1.reshape(1, _C))


def _tc_final(s1, cnt, r1):
    def body(s_ref, c_ref, r_ref, o_ref, g_ref):
        s = s_ref[0] + s_ref[1]
        o = s / jnp.maximum(c_ref[...], 1.0) + r_ref[...]
        o_ref[...] = o

        @pl.when(pl.program_id(0) == 0)
        def _():
            g_ref[...] = jnp.zeros_like(g_ref)
        g_ref[...] += jnp.sum(o, axis=0, keepdims=True) * (1.0 / _N)
    bn = 2000
    return pl.pallas_call(
        body,
        grid=(_N // bn,),
        in_specs=[pl.BlockSpec((_NC, bn, _C), lambda i: (0, i, 0)),
                  pl.BlockSpec((bn, 1), lambda i: (i, 0)),
                  pl.BlockSpec((bn, _C), lambda i: (i, 0))],
        out_specs=[pl.BlockSpec((bn, _C), lambda i: (i, 0)),
                   pl.BlockSpec((1, _C), lambda i: (0, 0))],
        out_shape=[jax.ShapeDtypeStruct((_N, _C), jnp.float32),
                   jax.ShapeDtypeStruct((1, _C), jnp.float32)],
    )(s1, cnt, r1)


def kernel(x, edge_index, Wl0, bl0, Wr0, Wl1, bl1, Wr1):
    z0 = jnp.zeros((_N, _H), jnp.float32)
    z1 = jnp.zeros((_N, _C), jnp.float32)
    zn = jnp.zeros((_N,), jnp.float32)

    p0 = _tc_proj(x, Wl0)
    p0f = p0.reshape(_NC * _N, _H)
    s0, cnt = _sc_pass_cached(_H, True)(p0f, edge_index, z0, zn)
    cnt = cnt.reshape(_N, 1)
    q1, r1 = _tc_mid(s0, cnt, x, Wr0, bl0, Wl1, Wr1, bl1)
    s1 = _sc_pass_cached(_C, False)(q1, edge_index, z1)
    if isinstance(s1, (list, tuple)):
        (s1,) = s1
    out, g = _tc_final(s1, cnt, r1)
    return (out, g)
